# Initial kernel scaffold; baseline (speedup 1.0000x reference)
import jax
import jax.numpy as jnp
from jax.experimental import pallas as pl


def kernel(inputs, embedding_weight):
    raise NotImplementedError("write your pallas kernel here")


# speedup vs baseline: 1.0957x; 1.0957x over previous
import functools

import jax
import jax.numpy as jnp
from jax.experimental import pallas as pl
from jax.experimental.pallas import tpu as pltpu

B, T, D = 32, 1024, 256
K = 1024
N = B * T
COMMITMENT_COST = 0.25

BM = 512
GRID = N // BM


def _vq_body(z_ref, zsq_ref, esq_ref, e_ref, enc_ref, qst_ref, loss_ref):
    z = z_ref[...]
    mm = jax.lax.dot_general(
        z, e_ref[...], (((1,), (1,)), ((), ())),
        preferred_element_type=jnp.float32,
    )
    dist = (zsq_ref[...] + esq_ref[...]) - 2.0 * mm
    idx = jnp.argmin(dist, axis=1)
    iota = jax.lax.broadcasted_iota(jnp.int32, (BM, K), 1)
    enc = (iota == idx[:, None]).astype(jnp.float32)
    enc_ref[...] = enc
    q = jax.lax.dot_general(
        enc, e_ref[...], (((1,), (0,)), ((), ())),
        preferred_element_type=jnp.float32,
    )
    qst_ref[...] = z + (q - z)
    partial = jnp.sum((q - z) ** 2)

    @pl.when(pl.program_id(0) == 0)
    def _():
        loss_ref[0, 0] = 0.0

    loss_ref[0, 0] += partial


@jax.jit
def kernel(inputs, embedding_weight):
    flat = inputs.reshape(N, D)
    zsq = jnp.sum(flat ** 2, axis=1, keepdims=True)
    esq = jnp.sum(embedding_weight ** 2, axis=1)[None, :]

    enc, qst, loss_sum = pl.pallas_call(
        _vq_body,
        grid=(GRID,),
        in_specs=[
            pl.BlockSpec((BM, D), lambda i: (i, 0)),
            pl.BlockSpec((BM, 1), lambda i: (i, 0)),
            pl.BlockSpec((1, K), lambda i: (0, 0)),
            pl.BlockSpec((K, D), lambda i: (0, 0)),
        ],
        out_specs=[
            pl.BlockSpec((BM, K), lambda i: (i, 0)),
            pl.BlockSpec((BM, D), lambda i: (i, 0)),
            pl.BlockSpec(memory_space=pltpu.SMEM),
        ],
        out_shape=[
            jax.ShapeDtypeStruct((N, K), jnp.float32),
            jax.ShapeDtypeStruct((N, D), jnp.float32),
            jax.ShapeDtypeStruct((1, 1), jnp.float32),
        ],
    )(flat, zsq, esq, embedding_weight)

    loss = COMMITMENT_COST * (loss_sum[0, 0] / (N * D))
    return qst.reshape(inputs.shape), loss, enc


# speedup vs baseline: 1.0963x; 1.0005x over previous
import functools

import jax
import jax.numpy as jnp
from jax.experimental import pallas as pl
from jax.experimental.pallas import tpu as pltpu

B, T, D = 32, 1024, 256
K = 1024
N = B * T
COMMITMENT_COST = 0.25

BM = 512
GRID = N // BM


def _vq_body(z_ref, zsq_ref, esq_ref, e2_ref, e_ref, enc_ref, qst_ref, loss_ref):
    z = z_ref[...]
    mm2 = jax.lax.dot_general(
        z, e2_ref[...], (((1,), (1,)), ((), ())),
        preferred_element_type=jnp.float32,
    )
    dist = (zsq_ref[...] + esq_ref[...]) - mm2
    idx = jnp.argmin(dist, axis=1)
    iota = jax.lax.broadcasted_iota(jnp.int32, (BM, K), 1)
    enc = (iota == idx[:, None]).astype(jnp.float32)
    enc_ref[...] = enc
    q = jax.lax.dot_general(
        enc, e_ref[...], (((1,), (0,)), ((), ())),
        preferred_element_type=jnp.float32,
    )
    qst_ref[...] = z + (q - z)
    loss_ref[0, 0, 0] = jnp.sum(jnp.min(dist, axis=1))


@jax.jit
def kernel(inputs, embedding_weight):
    flat = inputs.reshape(N, D)
    zsq = jnp.sum(flat ** 2, axis=1, keepdims=True)
    esq = jnp.sum(embedding_weight ** 2, axis=1)[None, :]
    e2 = embedding_weight * 2.0

    enc, qst, loss_parts = pl.pallas_call(
        _vq_body,
        grid=(GRID,),
        in_specs=[
            pl.BlockSpec((BM, D), lambda i: (i, 0)),
            pl.BlockSpec((BM, 1), lambda i: (i, 0)),
            pl.BlockSpec((1, K), lambda i: (0, 0)),
            pl.BlockSpec((K, D), lambda i: (0, 0)),
            pl.BlockSpec((K, D), lambda i: (0, 0)),
        ],
        out_specs=[
            pl.BlockSpec((BM, K), lambda i: (i, 0)),
            pl.BlockSpec((BM, D), lambda i: (i, 0)),
            pl.BlockSpec((1, 1, 1), lambda i: (i, 0, 0), memory_space=pltpu.SMEM),
        ],
        out_shape=[
            jax.ShapeDtypeStruct((N, K), jnp.float32),
            jax.ShapeDtypeStruct((N, D), jnp.float32),
            jax.ShapeDtypeStruct((GRID, 1, 1), jnp.float32),
        ],
        compiler_params=pltpu.CompilerParams(
            dimension_semantics=("parallel",),
        ),
    )(flat, zsq, esq, e2, embedding_weight)

    loss = COMMITMENT_COST * (jnp.sum(loss_parts) / (N * D))
    return qst.reshape(inputs.shape), loss, enc
